# baseline (speedup 1.0000x reference)
import functools

import jax
import jax.numpy as jnp
from jax import lax
from jax.experimental import pallas as pl
from jax.experimental.pallas import tpu as pltpu
from jax.experimental.pallas import tpu_sc as plsc

_NUM_CORES = 2
_NUM_SUBCORES = 16
_NUM_WORKERS = _NUM_CORES * _NUM_SUBCORES

_CHUNK = 128


@functools.partial(jax.jit, static_argnames=("n_rows", "embed_dim"))
def _sc_gather(idx2d, table, *, n_rows, embed_dim):
    n_chunks = idx2d.shape[0]
    chunks_per_w = n_chunks // _NUM_WORKERS

    mesh = plsc.VectorSubcoreMesh(
        core_axis_name="c", subcore_axis_name="s")

    @functools.partial(
        pl.kernel,
        out_type=jax.ShapeDtypeStruct((n_rows, embed_dim), jnp.float32),
        mesh=mesh,
        scratch_types=[
            pltpu.VMEM((_CHUNK,), jnp.int32),
            pltpu.VMEM((_CHUNK, embed_dim), jnp.float32),
            pltpu.SemaphoreType.DMA,
        ],
        compiler_params=pltpu.CompilerParams(use_tc_tiling_on_sc=False),
    )
    def k(idx_hbm, table_hbm, out_hbm, idx_v, rows_v, sem):
        wid = lax.axis_index("s") * _NUM_CORES + lax.axis_index("c")
        base = wid * chunks_per_w

        def body(i, carry):
            r = base + i
            pltpu.sync_copy(idx_hbm.at[r], idx_v)
            pltpu.async_copy(table_hbm.at[idx_v], rows_v, sem).wait()
            pltpu.sync_copy(rows_v, out_hbm.at[pl.ds(r * _CHUNK, _CHUNK)])
            return carry

        lax.fori_loop(0, chunks_per_w, body, 0)

    return k(idx2d, table)


def kernel(inputs, table):
    batch, seq = inputs.shape
    vocab, embed_dim = table.shape
    n_rows = batch * seq
    idx2d = inputs.reshape(n_rows // _CHUNK, _CHUNK).astype(jnp.int32)
    out = _sc_gather(idx2d, table, n_rows=n_rows, embed_dim=embed_dim)
    return out.reshape(batch, seq, embed_dim)


# speedup vs baseline: 1.1963x; 1.1963x over previous
import functools

import jax
import jax.numpy as jnp
from jax import lax
from jax.experimental import pallas as pl
from jax.experimental.pallas import tpu as pltpu
from jax.experimental.pallas import tpu_sc as plsc

_NUM_CORES = 2
_NUM_SUBCORES = 16
_NUM_WORKERS = _NUM_CORES * _NUM_SUBCORES

_CHUNK = 128
_NBUF = 8


@functools.partial(jax.jit, static_argnames=("n_rows", "embed_dim"))
def _sc_gather(idx2d, table, *, n_rows, embed_dim):
    n_chunks = idx2d.shape[0]
    chunks_per_w = n_chunks // _NUM_WORKERS
    rounds = chunks_per_w // _NBUF

    mesh = plsc.VectorSubcoreMesh(
        core_axis_name="c", subcore_axis_name="s")

    @functools.partial(
        pl.kernel,
        out_type=jax.ShapeDtypeStruct((n_rows, embed_dim), jnp.float32),
        mesh=mesh,
        scratch_types=[
            pltpu.VMEM((chunks_per_w, _CHUNK), jnp.int32),
            pltpu.VMEM((_NBUF, _CHUNK, embed_dim), jnp.float32),
            pltpu.SemaphoreType.DMA((_NBUF,)),
            pltpu.SemaphoreType.DMA((_NBUF,)),
        ],
        compiler_params=pltpu.CompilerParams(use_tc_tiling_on_sc=False),
    )
    def k(idx_hbm, table_hbm, out_hbm, idx_v, rows_v, gsem, wsem):
        wid = lax.axis_index("s") * _NUM_CORES + lax.axis_index("c")
        base = wid * chunks_per_w

        pltpu.sync_copy(idx_hbm.at[pl.ds(base, chunks_per_w)], idx_v)

        def gather(i, b):
            pltpu.async_copy(
                table_hbm.at[idx_v.at[i]], rows_v.at[b], gsem.at[b])

        def gather_wait(i, b):
            pltpu.make_async_copy(
                table_hbm.at[idx_v.at[i]], rows_v.at[b], gsem.at[b]).wait()

        def writeback(i, b):
            dst = out_hbm.at[pl.ds((base + i) * _CHUNK, _CHUNK)]
            pltpu.async_copy(rows_v.at[b], dst, wsem.at[b])
            return dst

        def writeback_wait(i, b):
            dst = out_hbm.at[pl.ds((base + i) * _CHUNK, _CHUNK)]
            pltpu.make_async_copy(rows_v.at[b], dst, wsem.at[b]).wait()

        for b in range(_NBUF):
            gather(b, b)

        def body(j, carry):
            for b in range(_NBUF):
                i = j * _NBUF + b
                gather_wait(i, b)
                writeback(i, b)

                @pl.when(j + 1 < rounds)
                def _():
                    writeback_wait(i, b)
                    gather(i + _NBUF, b)

            return carry

        lax.fori_loop(0, rounds, body, 0)

        for b in range(_NBUF):
            writeback_wait((rounds - 1) * _NBUF + b, b)

    return k(idx2d, table)


def kernel(inputs, table):
    batch, seq = inputs.shape
    vocab, embed_dim = table.shape
    n_rows = batch * seq
    idx2d = inputs.reshape(n_rows // _CHUNK, _CHUNK).astype(jnp.int32)
    out = _sc_gather(idx2d, table, n_rows=n_rows, embed_dim=embed_dim)
    return out.reshape(batch, seq, embed_dim)


# speedup vs baseline: 1.1971x; 1.0007x over previous
import functools

import jax
import jax.numpy as jnp
from jax import lax
from jax.experimental import pallas as pl
from jax.experimental.pallas import tpu as pltpu
from jax.experimental.pallas import tpu_sc as plsc

_NUM_CORES = 2
_NUM_SUBCORES = 16
_NUM_WORKERS = _NUM_CORES * _NUM_SUBCORES

_CHUNK = 128
_BLK = 2
_NBUF = 6
_LEAD = 3


@functools.partial(jax.jit, static_argnames=("n_rows", "embed_dim"))
def _sc_gather(idx2d, table, *, n_rows, embed_dim):
    n_chunks = idx2d.shape[0]
    chunks_per_w = n_chunks // _NUM_WORKERS
    blocks = chunks_per_w // _BLK

    mesh = plsc.VectorSubcoreMesh(
        core_axis_name="c", subcore_axis_name="s")

    @functools.partial(
        pl.kernel,
        out_type=jax.ShapeDtypeStruct((n_rows, embed_dim), jnp.float32),
        mesh=mesh,
        scratch_types=[
            pltpu.VMEM((chunks_per_w, _CHUNK), jnp.int32),
            pltpu.VMEM((_NBUF, _BLK * _CHUNK, embed_dim), jnp.float32),
            pltpu.SemaphoreType.DMA((_NBUF,)),
            pltpu.SemaphoreType.DMA((_NBUF,)),
        ],
        compiler_params=pltpu.CompilerParams(use_tc_tiling_on_sc=False),
    )
    def k(idx_hbm, table_hbm, out_hbm, idx_v, rows_v, gsem, wsem):
        wid = lax.axis_index("s") * _NUM_CORES + lax.axis_index("c")
        base = wid * chunks_per_w

        pltpu.sync_copy(idx_hbm.at[pl.ds(base, chunks_per_w)], idx_v)

        def gather(blk, b):
            for kk in range(_BLK):
                pltpu.async_copy(
                    table_hbm.at[idx_v.at[blk * _BLK + kk]],
                    rows_v.at[b, pl.ds(kk * _CHUNK, _CHUNK)],
                    gsem.at[b])

        def gather_wait(blk, b):
            for kk in range(_BLK):
                pltpu.make_async_copy(
                    table_hbm.at[idx_v.at[blk * _BLK + kk]],
                    rows_v.at[b, pl.ds(kk * _CHUNK, _CHUNK)],
                    gsem.at[b]).wait()

        def writeback(blk, b):
            dst = out_hbm.at[pl.ds((base + blk * _BLK) * _CHUNK,
                                   _BLK * _CHUNK)]
            pltpu.async_copy(rows_v.at[b], dst, wsem.at[b])

        def writeback_wait(blk, b):
            dst = out_hbm.at[pl.ds((base + blk * _BLK) * _CHUNK,
                                   _BLK * _CHUNK)]
            pltpu.make_async_copy(rows_v.at[b], dst, wsem.at[b]).wait()

        for blk in range(_LEAD):
            gather(blk, blk)

        def body(B, carry):
            nxt = B + _LEAD

            @pl.when(nxt < blocks)
            def _():
                bn = lax.rem(nxt, _NBUF)

                @pl.when(nxt >= _NBUF)
                def _():
                    writeback_wait(nxt - _NBUF, bn)

                gather(nxt, bn)

            b = lax.rem(B, _NBUF)
            gather_wait(B, b)
            writeback(B, b)
            return carry

        lax.fori_loop(0, blocks, body, 0)

        for blk in range(blocks - _NBUF, blocks):
            writeback_wait(blk, blk % _NBUF)

    return k(idx2d, table)


def kernel(inputs, table):
    batch, seq = inputs.shape
    vocab, embed_dim = table.shape
    n_rows = batch * seq
    idx2d = inputs.reshape(n_rows // _CHUNK, _CHUNK).astype(jnp.int32)
    out = _sc_gather(idx2d, table, n_rows=n_rows, embed_dim=embed_dim)
    return out.reshape(batch, seq, embed_dim)
